# baseline (speedup 1.0000x reference)
import functools

import jax
import jax.numpy as jnp
from jax import lax
from jax.experimental import pallas as pl
from jax.experimental.pallas import tpu as pltpu
from jax.experimental.pallas import tpu_sc as plsc

DIM = 64
MAXF = 432000
B_TOTAL = 1024 * 200
L = 16
NC, NS = 2, 16
NW = NC * NS
SUB = 128
ROWS_PER_W = B_TOTAL // NW
NSUB = ROWS_PER_W // SUB

_INV60 = 1.0 / 60.0

_mesh = plsc.VectorSubcoreMesh(core_axis_name="c", subcore_axis_name="s")


@functools.partial(
    pl.kernel,
    mesh=_mesh,
    compiler_params=pltpu.CompilerParams(use_tc_tiling_on_sc=False),
    out_type=jax.ShapeDtypeStruct((B_TOTAL, DIM), jnp.bfloat16),
    scratch_types=[
        pltpu.VMEM((NSUB, SUB), jnp.int32),
        pltpu.VMEM((NSUB, SUB), jnp.int32),
        pltpu.VMEM((2, SUB, DIM), jnp.bfloat16),
        pltpu.VMEM((2, SUB, DIM), jnp.bfloat16),
        pltpu.SemaphoreType.DMA,
        pltpu.SemaphoreType.DMA,
    ],
)
def _encode(idx_hbm, ftab, stab, out_hbm,
            idx_v, sidx_v, fbuf, sbuf, sem0, sem1):
    wid = lax.axis_index("s") * NC + lax.axis_index("c")
    row0 = wid * ROWS_PER_W
    sems = (sem0, sem1)

    pltpu.sync_copy(idx_hbm.at[wid], idx_v)

    def derive(j, carry):
        for k in range(SUB // L):
            s = pl.ds(k * L, L)
            f = idx_v[j, s].astype(jnp.float32)
            sidx_v[j, s] = (f * _INV60).astype(jnp.int32)
        return carry

    lax.fori_loop(0, NSUB, derive, 0)

    def gathers(j, b):
        return [
            pltpu.make_async_copy(ftab.at[idx_v.at[j]], fbuf.at[b], sems[b]),
            pltpu.make_async_copy(stab.at[sidx_v.at[j]], sbuf.at[b], sems[b]),
        ]

    def fire(j, b):
        for cp in gathers(j, b):
            cp.start()

    fire(0, 0)

    def pair(i, carry):
        j0 = 2 * i
        for b in range(2):
            j = j0 + b

            @pl.when(j < NSUB - 1)
            def _():
                fire(j + 1, 1 - b)

            for cp in gathers(j, b):
                cp.wait()

            def add_rows(r4, c2):
                for rr in range(4):
                    r = r4 * 4 + rr
                    for q in range(DIM // (2 * L)):
                        s = pl.ds(q * 2 * L, 2 * L)
                        fbuf[b, r, s] = fbuf[b, r, s] + sbuf[b, r, s]
                return c2

            lax.fori_loop(0, SUB // 4, add_rows, 0)
            pltpu.sync_copy(fbuf.at[b], out_hbm.at[pl.ds(row0 + j * SUB, SUB)])
        return carry

    lax.fori_loop(0, NSUB // 2, pair, 0)


def kernel(frame_indices, frame_table, second_table, minute_table, pe):
    bsz, seq = frame_indices.shape
    idx = frame_indices.astype(jnp.int32).reshape(NW, NSUB, SUB)
    comb = (frame_table + pe).astype(jnp.bfloat16)
    small = (second_table
             + jnp.repeat(minute_table, 60, axis=0)).astype(jnp.bfloat16)
    out = _encode(idx, comb, small)
    return out.astype(jnp.float32).reshape(bsz, seq, DIM)


# speedup vs baseline: 1.1997x; 1.1997x over previous
import functools

import jax
import jax.numpy as jnp
from jax import lax
from jax.experimental import pallas as pl
from jax.experimental.pallas import tpu as pltpu
from jax.experimental.pallas import tpu_sc as plsc

DIM = 64
MAXF = 432000
B_TOTAL = 1024 * 200
L = 16
NC, NS = 2, 16
NW = NC * NS
SUB = 128
ROWS_PER_W = B_TOTAL // NW
NSUB = ROWS_PER_W // SUB

_INV60 = 1.0 / 60.0

_mesh = plsc.VectorSubcoreMesh(core_axis_name="c", subcore_axis_name="s")


@functools.partial(
    pl.kernel,
    mesh=_mesh,
    compiler_params=pltpu.CompilerParams(use_tc_tiling_on_sc=False),
    out_type=jax.ShapeDtypeStruct((B_TOTAL, DIM), jnp.float32),
    scratch_types=[
        pltpu.VMEM((NSUB, SUB), jnp.int32),
        pltpu.VMEM((NSUB, SUB), jnp.int32),
        pltpu.VMEM((2, SUB, DIM), jnp.float32),
        pltpu.VMEM((2, SUB, DIM), jnp.float32),
        pltpu.SemaphoreType.DMA,
        pltpu.SemaphoreType.DMA,
    ],
)
def _encode(idx_hbm, ftab, stab, out_hbm,
            idx_v, sidx_v, fbuf, sbuf, sem0, sem1):
    wid = lax.axis_index("s") * NC + lax.axis_index("c")
    row0 = wid * ROWS_PER_W
    sems = (sem0, sem1)

    pltpu.sync_copy(idx_hbm.at[wid], idx_v)

    def derive(j, carry):
        for k in range(SUB // L):
            s = pl.ds(k * L, L)
            f = idx_v[j, s].astype(jnp.float32)
            sidx_v[j, s] = (f * _INV60).astype(jnp.int32)
        return carry

    lax.fori_loop(0, NSUB, derive, 0)

    def gathers(j, b):
        return [
            pltpu.make_async_copy(ftab.at[idx_v.at[j]], fbuf.at[b], sems[b]),
            pltpu.make_async_copy(stab.at[sidx_v.at[j]], sbuf.at[b], sems[b]),
        ]

    def fire(j, b):
        for cp in gathers(j, b):
            cp.start()

    fire(0, 0)

    def pair(i, carry):
        j0 = 2 * i
        for b in range(2):
            j = j0 + b

            @pl.when(j < NSUB - 1)
            def _():
                fire(j + 1, 1 - b)

            for cp in gathers(j, b):
                cp.wait()

            def add_rows(r4, c2):
                for rr in range(4):
                    r = r4 * 4 + rr
                    for q in range(DIM // L):
                        s = pl.ds(q * L, L)
                        fbuf[b, r, s] = fbuf[b, r, s] + sbuf[b, r, s]
                return c2

            lax.fori_loop(0, SUB // 4, add_rows, 0)
            pltpu.sync_copy(fbuf.at[b], out_hbm.at[pl.ds(row0 + j * SUB, SUB)])
        return carry

    lax.fori_loop(0, NSUB // 2, pair, 0)


def kernel(frame_indices, frame_table, second_table, minute_table, pe):
    bsz, seq = frame_indices.shape
    idx = frame_indices.astype(jnp.int32).reshape(NW, NSUB, SUB)
    comb = frame_table + pe
    small = second_table + jnp.repeat(minute_table, 60, axis=0)
    out = _encode(idx, comb, small)
    return out.reshape(bsz, seq, DIM)
